# baseline (speedup 1.0000x reference)
import math

import jax
import jax.numpy as jnp
from jax import lax
from jax.experimental import pallas as pl
from jax.experimental.pallas import tpu as pltpu
from jax.experimental.pallas import tpu_sc as plsc

N = 10000
E = 320000
H = 128
INV_SQRT_3 = 1.0 / math.sqrt(3.0)
INV_SQRT_H = 1.0 / math.sqrt(float(H))
SILU_SCALE = 1.0 / 0.6

NC, NS = 2, 16
NW = NC * NS
EPW = E // NW
C = 40
NCH = EPW // C
NPAD = 10240
RPS = NPAD // NS
ZR = C
NZ = RPS // ZR


def _pack_pair(hi, lo):
    hb = lax.bitcast_convert_type(hi, jnp.uint32)
    lb = lax.bitcast_convert_type(lo, jnp.uint32)
    hb = (hb + jnp.uint32(0x8000)) & jnp.uint32(0xFFFF0000)
    lb = (lb + jnp.uint32(0x8000)) >> jnp.uint32(16)
    return lax.bitcast_convert_type(hb | lb, jnp.int32)


def _xh_body(x_ref, vec_ref, w1t_ref, b1_ref, w2t_ref, b2_ref,
             a_ref, o3_ref, v0_ref, v1_ref, v2_ref):
    h = jnp.dot(x_ref[...], w1t_ref[...], preferred_element_type=jnp.float32)
    h = h + b1_ref[...]
    h = h * jax.nn.sigmoid(h) * SILU_SCALE
    o = jnp.dot(h, w2t_ref[...], preferred_element_type=jnp.float32) + b2_ref[...]
    a_ref[...] = _pack_pair(o[:, :H], o[:, H:2 * H])
    o3_ref[...] = lax.bitcast_convert_type(o[:, 2 * H:], jnp.int32)
    v0_ref[...] = vec_ref[:, 0, :]
    v1_ref[...] = vec_ref[:, 1, :]
    v2_ref[...] = vec_ref[:, 2, :]


def _compute_tables(x, vec, w1t, b1r, w2t, b2r):
    nb = 10
    bs = N // nb
    return pl.pallas_call(
        _xh_body,
        grid=(nb,),
        in_specs=[
            pl.BlockSpec((bs, H), lambda n: (n, 0)),
            pl.BlockSpec((bs, 3, H), lambda n: (n, 0, 0)),
            pl.BlockSpec((H, H // 2), lambda n: (0, 0)),
            pl.BlockSpec((1, H // 2), lambda n: (0, 0)),
            pl.BlockSpec((H // 2, 3 * H), lambda n: (0, 0)),
            pl.BlockSpec((1, 3 * H), lambda n: (0, 0)),
        ],
        out_specs=[pl.BlockSpec((bs, H), lambda n: (n, 0))] * 5,
        out_shape=[jax.ShapeDtypeStruct((N, H), jnp.int32)] * 2
        + [jax.ShapeDtypeStruct((N, H), jnp.float32)] * 3,
    )(x, vec, w1t, b1r, w2t, b2r)


def _rbf_body(rbf_ref, wet_ref, be_ref, ev_ref,
              b_ref, r2_ref, e0_ref, e1_ref, e2_ref):
    o = jnp.dot(rbf_ref[...], wet_ref[...], preferred_element_type=jnp.float32)
    o = o + be_ref[...]
    b_ref[...] = _pack_pair(o[:, :H], o[:, H:2 * H])
    r2_ref[...] = lax.bitcast_convert_type(o[:, 2 * H:], jnp.int32)
    e0_ref[...] = jnp.broadcast_to(ev_ref[:, 0][:, None], e0_ref.shape)
    e1_ref[...] = jnp.broadcast_to(ev_ref[:, 1][:, None], e1_ref.shape)
    e2_ref[...] = jnp.broadcast_to(ev_ref[:, 2][:, None], e2_ref.shape)


def _compute_edge_arrays(edge_rbf, wet, ber, edge_vector):
    nb = 80
    bs = E // nb
    f = edge_rbf.shape[1]
    return pl.pallas_call(
        _rbf_body,
        grid=(nb,),
        in_specs=[
            pl.BlockSpec((bs, f), lambda n: (n, 0)),
            pl.BlockSpec((f, 3 * H), lambda n: (0, 0)),
            pl.BlockSpec((1, 3 * H), lambda n: (0, 0)),
            pl.BlockSpec((bs, 3), lambda n: (n, 0)),
        ],
        out_specs=[pl.BlockSpec((bs, H), lambda n: (n, 0))] * 2
        + [pl.BlockSpec((bs, 16), lambda n: (n, 0))] * 3,
        out_shape=[jax.ShapeDtypeStruct((E, H), jnp.int32)] * 2
        + [jax.ShapeDtypeStruct((E, 16), jnp.float32)] * 3,
    )(edge_rbf, wet, ber, edge_vector)


def _zero_acc(zv, acc_sh, sid):
    def zrow(e, _):
        for f in range(H // 16):
            zv[e, pl.ds(f * 16, 16)] = jnp.zeros((16,), jnp.float32)
        return 0
    lax.fori_loop(0, ZR, zrow, 0)

    def zcopy(t, _):
        pltpu.sync_copy(zv, acc_sh.at[pl.ds(sid * RPS + t * ZR, ZR)])
        return 0
    lax.fori_loop(0, NZ, zcopy, 0)


def _writeout(zv, acc_sh, out_hbm, cid, sid):
    def wcopy(t, _):
        r0 = sid * RPS + t * ZR
        pltpu.sync_copy(acc_sh.at[pl.ds(r0, ZR)], zv)
        pltpu.sync_copy(zv, out_hbm.at[cid, pl.ds(r0, ZR)])
        return 0
    lax.fori_loop(0, NZ, wcopy, 0)


MASK_HI = -65536


def _unpack_hi(v):
    return lax.bitcast_convert_type(v & MASK_HI, jnp.float32)


def _unpack_lo(v):
    return lax.bitcast_convert_type(v << 16, jnp.float32)


def _f32(v):
    return lax.bitcast_convert_type(v, jnp.float32)


def _sc_body(apk_h, xh3_h, v0_h, v1_h, v2_h, bpk_h, rb2_h, e0_h, e1_h, e2_h,
             j_h, i_h, px_h, pv_h,
             acc_sh, jv4, iv4, avs, vvs, bvs, evs, sem0, sem1, semi0, semi1):
    cid = lax.axis_index("c")
    sid = lax.axis_index("s")
    wid = sid * NC + cid
    base = wid * EPW
    sems = (sem0, sem1)
    semi = (semi0, semi1)
    zv = vvs.at[0]

    def run_phase(a_h, v_h, b_h, e_h, is_dx, wout):
        _zero_acc(zv, acc_sh, sid)
        plsc.subcore_barrier()

        def fire_idx(t, par):
            d4 = lax.rem(t, 4)
            pltpu.async_copy(j_h.at[pl.ds(base + t * C, C)], jv4.at[d4],
                             semi[par])
            pltpu.async_copy(i_h.at[pl.ds(base + t * C, C)], iv4.at[d4],
                             semi[par])

        def drain_idx(par):
            pltpu.make_async_copy(j_h.at[pl.ds(0, C)], jv4.at[0],
                                  semi[par]).wait()
            pltpu.make_async_copy(i_h.at[pl.ds(0, C)], iv4.at[0],
                                  semi[par]).wait()

        def fire_data(t, b):
            jref = jv4.at[lax.rem(t, 4)]
            off = base + t * C
            pltpu.async_copy(a_h.at[jref], avs.at[b], sems[b])
            pltpu.async_copy(b_h.at[pl.ds(off, C)], bvs.at[b], sems[b])
            if not is_dx:
                pltpu.async_copy(v_h.at[jref], vvs.at[b], sems[b])
                pltpu.async_copy(e_h.at[pl.ds(off, C)], evs.at[b], sems[b])

        def drain_data(b):
            pltpu.make_async_copy(a_h.at[jv4.at[0]], avs.at[b],
                                  sems[b]).wait()
            pltpu.make_async_copy(b_h.at[pl.ds(0, C)], bvs.at[b],
                                  sems[b]).wait()
            if not is_dx:
                pltpu.make_async_copy(v_h.at[jv4.at[0]], vvs.at[b],
                                      sems[b]).wait()
                pltpu.make_async_copy(e_h.at[pl.ds(0, C)], evs.at[b],
                                      sems[b]).wait()

        fire_idx(0, 0)
        fire_idx(1, 1)
        drain_idx(0)
        fire_data(0, 0)
        fire_idx(2, 0)
        drain_idx(1)
        fire_data(1, 1)
        fire_idx(3, 1)

        def step(t, b):
            drain_data(b)
            av_b = avs.at[b]
            vv_b = vvs.at[b]
            bv_b = bvs.at[b]
            ev_b = evs.at[b]

            if is_dx:
                def edge(e, _):
                    for f in range(H // 16):
                        s = pl.ds(f * 16, 16)
                        vv_b[e, s] = _f32(av_b[e, s]) * _f32(bv_b[e, s])
                    return 0
            else:
                def edge(e, _):
                    evr = ev_b[e, pl.ds(0, 16)]
                    for f in range(H // 16):
                        s = pl.ds(f * 16, 16)
                        pa = av_b[e, s]
                        pb = bv_b[e, s]
                        vv_b[e, s] = (_unpack_hi(pa) * _unpack_hi(pb)
                                      * vv_b[e, s]
                                      + _unpack_lo(pa) * _unpack_lo(pb) * evr)
                    return 0
            lax.fori_loop(0, C, edge, 0, unroll=4)
            pltpu.sync_copy(vv_b, acc_sh.at[iv4.at[lax.rem(t, 4)]], add=True)

            @pl.when(t + 2 < NCH)
            def _():
                drain_idx(b)
                fire_data(t + 2, b)

            @pl.when(t + 4 < NCH)
            def _():
                fire_idx(t + 4, b)

        def chunk2(t2, _):
            step(t2 * 2, 0)
            step(t2 * 2 + 1, 1)
            return 0

        lax.fori_loop(0, NCH // 2, chunk2, 0)
        plsc.subcore_barrier()

        def wcopy(t, _):
            r0 = sid * RPS + t * ZR
            pltpu.sync_copy(acc_sh.at[pl.ds(r0, ZR)], zv)
            pltpu.sync_copy(zv, wout(r0))
            return 0
        lax.fori_loop(0, NZ, wcopy, 0)
        plsc.subcore_barrier()

    run_phase(xh3_h, None, rb2_h, None, True,
              lambda r0: px_h.at[cid, pl.ds(r0, ZR)])
    for d, (vh, eh) in enumerate([(v0_h, e0_h), (v1_h, e1_h), (v2_h, e2_h)]):
        run_phase(apk_h, vh, bpk_h, eh, False,
                  lambda r0, d=d: pv_h.at[d, cid, pl.ds(r0, ZR)])


def _sc_passes(apk, xh3, v0, v1, v2, bpk, rb2, e0, e1, e2, j, i):
    mesh = plsc.VectorSubcoreMesh(core_axis_name="c", subcore_axis_name="s")
    return pl.kernel(
        _sc_body,
        out_type=[
            jax.ShapeDtypeStruct((NC, NPAD, H), jnp.float32),
            jax.ShapeDtypeStruct((3, NC, NPAD, H), jnp.float32),
        ],
        mesh=mesh,
        scratch_types=[
            pltpu.VMEM_SHARED((NPAD, H), jnp.float32),
            pltpu.VMEM((4, C), jnp.int32),
            pltpu.VMEM((4, C), jnp.int32),
            pltpu.VMEM((2, C, H), jnp.int32),
            pltpu.VMEM((2, C, H), jnp.float32),
            pltpu.VMEM((2, C, H), jnp.int32),
            pltpu.VMEM((2, C, 16), jnp.float32),
            pltpu.SemaphoreType.DMA,
            pltpu.SemaphoreType.DMA,
            pltpu.SemaphoreType.DMA,
            pltpu.SemaphoreType.DMA,
        ],
    )(apk, xh3, v0, v1, v2, bpk, rb2, e0, e1, e2, j, i)


def _combine_body(px_ref, pv_ref, dx_ref, dv_ref):
    dx_ref[...] = px_ref[0] + px_ref[1]
    dv_ref[...] = jnp.concatenate(
        [(pv_ref[0, 0] + pv_ref[0, 1])[:, None, :],
         (pv_ref[1, 0] + pv_ref[1, 1])[:, None, :],
         (pv_ref[2, 0] + pv_ref[2, 1])[:, None, :]], axis=1)


def _combine(px, pv):
    nb = 25
    bs = N // nb
    return pl.pallas_call(
        _combine_body,
        grid=(nb,),
        in_specs=[
            pl.BlockSpec((NC, bs, H), lambda n: (0, n, 0)),
            pl.BlockSpec((3, NC, bs, H), lambda n: (0, 0, n, 0)),
        ],
        out_specs=[
            pl.BlockSpec((bs, H), lambda n: (n, 0)),
            pl.BlockSpec((bs, 3, H), lambda n: (n, 0, 0)),
        ],
        out_shape=[
            jax.ShapeDtypeStruct((N, H), jnp.float32),
            jax.ShapeDtypeStruct((N, 3, H), jnp.float32),
        ],
    )(px, pv)


def kernel(x, vec, edge_index, edge_rbf, edge_vector, W1, b1, W2, b2, We, be):
    f32 = jnp.float32
    j = edge_index[0]
    i = edge_index[1]

    s01 = INV_SQRT_3 * INV_SQRT_H
    scales = jnp.concatenate([
        jnp.full((2 * H,), s01, f32),
        jnp.full((H,), INV_SQRT_3, f32),
    ])
    wet = (We * scales[:, None]).T
    ber = (be * scales)[None, :]

    w1t = W1.T
    b1r = b1[None, :]
    w2t = W2.T
    b2r = b2[None, :]

    apk, xh3, v0, v1, v2 = _compute_tables(x, vec, w1t, b1r, w2t, b2r)
    bpk, rb2, e0, e1, e2 = _compute_edge_arrays(edge_rbf, wet, ber,
                                                edge_vector)

    px, pv = _sc_passes(apk, xh3, v0, v1, v2, bpk, rb2, e0, e1, e2, j, i)

    d_x, d_vec = _combine(px, pv)
    return (d_x, d_vec)


# speedup vs baseline: 1.8056x; 1.8056x over previous
import math

import jax
import jax.numpy as jnp
from jax import lax
from jax.experimental import pallas as pl
from jax.experimental.pallas import tpu as pltpu
from jax.experimental.pallas import tpu_sc as plsc

N = 10000
E = 320000
H = 128
INV_SQRT_3 = 1.0 / math.sqrt(3.0)
INV_SQRT_H = 1.0 / math.sqrt(float(H))
SILU_SCALE = 1.0 / 0.6

NC, NS = 2, 16
NW = NC * NS
EPW = E // NW
C = 40
NCH = EPW // C
NPAD = 10240
RPS = NPAD // NS
ZR = C
NZ = RPS // ZR


def _pack_pair(hi, lo):
    hb = lax.bitcast_convert_type(hi, jnp.uint32)
    lb = lax.bitcast_convert_type(lo, jnp.uint32)
    hb = (hb + jnp.uint32(0x8000)) & jnp.uint32(0xFFFF0000)
    lb = (lb + jnp.uint32(0x8000)) >> jnp.uint32(16)
    return lax.bitcast_convert_type(hb | lb, jnp.int32)


_DNUM = (((1,), (1,)), ((), ()))


def _xh_body(x_ref, vec_ref, w1_ref, b1_ref, w2_ref, b2_ref,
             a_ref, o3_ref, v0_ref, v1_ref, v2_ref):
    h = lax.dot_general(x_ref[...], w1_ref[...], _DNUM,
                        preferred_element_type=jnp.float32)
    h = h + b1_ref[...]
    h = h * jax.nn.sigmoid(h) * SILU_SCALE
    o = lax.dot_general(h, w2_ref[...], _DNUM,
                        preferred_element_type=jnp.float32) + b2_ref[...]
    a_ref[...] = _pack_pair(o[:, :H], o[:, H:2 * H])
    o3_ref[...] = lax.bitcast_convert_type(o[:, 2 * H:], jnp.int32)
    v0_ref[...] = vec_ref[:, 0, :]
    v1_ref[...] = vec_ref[:, 1, :]
    v2_ref[...] = vec_ref[:, 2, :]


def _compute_tables(x, vec, w1, b1r, w2, b2r):
    nb = 10
    bs = N // nb
    return pl.pallas_call(
        _xh_body,
        grid=(nb,),
        in_specs=[
            pl.BlockSpec((bs, H), lambda n: (n, 0)),
            pl.BlockSpec((bs, 3, H), lambda n: (n, 0, 0)),
            pl.BlockSpec((H // 2, H), lambda n: (0, 0)),
            pl.BlockSpec((1, H // 2), lambda n: (0, 0)),
            pl.BlockSpec((3 * H, H // 2), lambda n: (0, 0)),
            pl.BlockSpec((1, 3 * H), lambda n: (0, 0)),
        ],
        out_specs=[pl.BlockSpec((bs, H), lambda n: (n, 0))] * 5,
        out_shape=[jax.ShapeDtypeStruct((N, H), jnp.int32)] * 2
        + [jax.ShapeDtypeStruct((N, H), jnp.float32)] * 3,
    )(x, vec, w1, b1r, w2, b2r)


def _rbf_body(rbf_ref, we_ref, be_ref, ev_ref,
              b_ref, r2_ref, e0_ref, e1_ref, e2_ref):
    o = lax.dot_general(rbf_ref[...], we_ref[...], _DNUM,
                        preferred_element_type=jnp.float32)
    o = o + be_ref[...]
    s01 = INV_SQRT_3 * INV_SQRT_H
    o = o * jnp.concatenate([
        jnp.full((1, 2 * H), s01, jnp.float32),
        jnp.full((1, H), INV_SQRT_3, jnp.float32)], axis=1)
    b_ref[...] = _pack_pair(o[:, :H], o[:, H:2 * H])
    r2_ref[...] = lax.bitcast_convert_type(o[:, 2 * H:], jnp.int32)
    e0_ref[...] = jnp.broadcast_to(ev_ref[:, 0][:, None], e0_ref.shape)
    e1_ref[...] = jnp.broadcast_to(ev_ref[:, 1][:, None], e1_ref.shape)
    e2_ref[...] = jnp.broadcast_to(ev_ref[:, 2][:, None], e2_ref.shape)


def _compute_edge_arrays(edge_rbf, we, ber, edge_vector):
    nb = 80
    bs = E // nb
    f = edge_rbf.shape[1]
    return pl.pallas_call(
        _rbf_body,
        grid=(nb,),
        in_specs=[
            pl.BlockSpec((bs, f), lambda n: (n, 0)),
            pl.BlockSpec((3 * H, f), lambda n: (0, 0)),
            pl.BlockSpec((1, 3 * H), lambda n: (0, 0)),
            pl.BlockSpec((bs, 3), lambda n: (n, 0)),
        ],
        out_specs=[pl.BlockSpec((bs, H), lambda n: (n, 0))] * 2
        + [pl.BlockSpec((bs, 16), lambda n: (n, 0))] * 3,
        out_shape=[jax.ShapeDtypeStruct((E, H), jnp.int32)] * 2
        + [jax.ShapeDtypeStruct((E, 16), jnp.float32)] * 3,
    )(edge_rbf, we, ber, edge_vector)


def _zero_acc(zv, acc_sh, sid):
    def zrow(e, _):
        for f in range(H // 16):
            zv[e, pl.ds(f * 16, 16)] = jnp.zeros((16,), jnp.float32)
        return 0
    lax.fori_loop(0, ZR, zrow, 0)

    def zcopy(t, _):
        pltpu.sync_copy(zv, acc_sh.at[pl.ds(sid * RPS + t * ZR, ZR)])
        return 0
    lax.fori_loop(0, NZ, zcopy, 0)


def _writeout(zv, acc_sh, out_hbm, cid, sid):
    def wcopy(t, _):
        r0 = sid * RPS + t * ZR
        pltpu.sync_copy(acc_sh.at[pl.ds(r0, ZR)], zv)
        pltpu.sync_copy(zv, out_hbm.at[cid, pl.ds(r0, ZR)])
        return 0
    lax.fori_loop(0, NZ, wcopy, 0)


MASK_HI = -65536


def _unpack_hi(v):
    return lax.bitcast_convert_type(v & MASK_HI, jnp.float32)


def _unpack_lo(v):
    return lax.bitcast_convert_type(v << 16, jnp.float32)


def _f32(v):
    return lax.bitcast_convert_type(v, jnp.float32)


def _sc_body(apk_h, xh3_h, v0_h, v1_h, v2_h, bpk_h, rb2_h, e0_h, e1_h, e2_h,
             j_h, i_h, px_h, pv_h,
             acc_sh, jv4, iv4, avs, vvs, bvs, evs, sem0, sem1, semi0, semi1):
    cid = lax.axis_index("c")
    sid = lax.axis_index("s")
    wid = sid * NC + cid
    base = wid * EPW
    sems = (sem0, sem1)
    semi = (semi0, semi1)
    zv = vvs.at[0]

    def run_phase(a_h, v_h, b_h, e_h, is_dx, wout):
        _zero_acc(zv, acc_sh, sid)
        plsc.subcore_barrier()

        def fire_idx(t, par):
            d4 = lax.rem(t, 4)
            pltpu.async_copy(j_h.at[pl.ds(base + t * C, C)], jv4.at[d4],
                             semi[par])
            pltpu.async_copy(i_h.at[pl.ds(base + t * C, C)], iv4.at[d4],
                             semi[par])

        def drain_idx(par):
            pltpu.make_async_copy(j_h.at[pl.ds(0, C)], jv4.at[0],
                                  semi[par]).wait()
            pltpu.make_async_copy(i_h.at[pl.ds(0, C)], iv4.at[0],
                                  semi[par]).wait()

        def fire_data(t, b):
            jref = jv4.at[lax.rem(t, 4)]
            off = base + t * C
            pltpu.async_copy(a_h.at[jref], avs.at[b], sems[b])
            pltpu.async_copy(b_h.at[pl.ds(off, C)], bvs.at[b], sems[b])
            if not is_dx:
                pltpu.async_copy(v_h.at[jref], vvs.at[b], sems[b])
                pltpu.async_copy(e_h.at[pl.ds(off, C)], evs.at[b], sems[b])

        def drain_data(b):
            pltpu.make_async_copy(a_h.at[jv4.at[0]], avs.at[b],
                                  sems[b]).wait()
            pltpu.make_async_copy(b_h.at[pl.ds(0, C)], bvs.at[b],
                                  sems[b]).wait()
            if not is_dx:
                pltpu.make_async_copy(v_h.at[jv4.at[0]], vvs.at[b],
                                      sems[b]).wait()
                pltpu.make_async_copy(e_h.at[pl.ds(0, C)], evs.at[b],
                                      sems[b]).wait()

        fire_idx(0, 0)
        fire_idx(1, 1)
        drain_idx(0)
        fire_data(0, 0)
        fire_idx(2, 0)
        drain_idx(1)
        fire_data(1, 1)
        fire_idx(3, 1)

        def step(t, b):
            drain_data(b)
            av_b = avs.at[b]
            vv_b = vvs.at[b]
            bv_b = bvs.at[b]
            ev_b = evs.at[b]

            if is_dx:
                def edge(e, _):
                    for f in range(H // 16):
                        s = pl.ds(f * 16, 16)
                        vv_b[e, s] = _f32(av_b[e, s]) * _f32(bv_b[e, s])
                    return 0
            else:
                def edge(e, _):
                    evr = ev_b[e, pl.ds(0, 16)]
                    for f in range(H // 16):
                        s = pl.ds(f * 16, 16)
                        pa = av_b[e, s]
                        pb = bv_b[e, s]
                        vv_b[e, s] = (_unpack_hi(pa) * _unpack_hi(pb)
                                      * vv_b[e, s]
                                      + _unpack_lo(pa) * _unpack_lo(pb) * evr)
                    return 0
            lax.fori_loop(0, C, edge, 0)
            pltpu.sync_copy(vv_b, acc_sh.at[iv4.at[lax.rem(t, 4)]], add=True)

            @pl.when(t + 2 < NCH)
            def _():
                drain_idx(b)
                fire_data(t + 2, b)

            @pl.when(t + 4 < NCH)
            def _():
                fire_idx(t + 4, b)

        def chunk2(t2, _):
            step(t2 * 2, 0)
            step(t2 * 2 + 1, 1)
            return 0

        lax.fori_loop(0, NCH // 2, chunk2, 0)
        plsc.subcore_barrier()

        def wcopy(t, _):
            r0 = sid * RPS + t * ZR
            pltpu.sync_copy(acc_sh.at[pl.ds(r0, ZR)], zv)
            pltpu.sync_copy(zv, wout(r0))
            return 0
        lax.fori_loop(0, NZ, wcopy, 0)
        plsc.subcore_barrier()

    run_phase(xh3_h, None, rb2_h, None, True,
              lambda r0: px_h.at[cid, pl.ds(r0, ZR)])
    for d, (vh, eh) in enumerate([(v0_h, e0_h), (v1_h, e1_h), (v2_h, e2_h)]):
        run_phase(apk_h, vh, bpk_h, eh, False,
                  lambda r0, d=d: pv_h.at[d, cid, pl.ds(r0, ZR)])


def _sc_passes(apk, xh3, v0, v1, v2, bpk, rb2, e0, e1, e2, j, i):
    mesh = plsc.VectorSubcoreMesh(core_axis_name="c", subcore_axis_name="s")
    return pl.kernel(
        _sc_body,
        out_type=[
            jax.ShapeDtypeStruct((NC, NPAD, H), jnp.float32),
            jax.ShapeDtypeStruct((3, NC, NPAD, H), jnp.float32),
        ],
        mesh=mesh,
        scratch_types=[
            pltpu.VMEM_SHARED((NPAD, H), jnp.float32),
            pltpu.VMEM((4, C), jnp.int32),
            pltpu.VMEM((4, C), jnp.int32),
            pltpu.VMEM((2, C, H), jnp.int32),
            pltpu.VMEM((2, C, H), jnp.float32),
            pltpu.VMEM((2, C, H), jnp.int32),
            pltpu.VMEM((2, C, 16), jnp.float32),
            pltpu.SemaphoreType.DMA,
            pltpu.SemaphoreType.DMA,
            pltpu.SemaphoreType.DMA,
            pltpu.SemaphoreType.DMA,
        ],
    )(apk, xh3, v0, v1, v2, bpk, rb2, e0, e1, e2, j, i)


def _combine_body(px_ref, pv_ref, dx_ref, dv_ref):
    dx_ref[...] = px_ref[0] + px_ref[1]
    dv_ref[...] = jnp.concatenate(
        [(pv_ref[0, 0] + pv_ref[0, 1])[:, None, :],
         (pv_ref[1, 0] + pv_ref[1, 1])[:, None, :],
         (pv_ref[2, 0] + pv_ref[2, 1])[:, None, :]], axis=1)


def _combine(px, pv):
    nb = 25
    bs = N // nb
    return pl.pallas_call(
        _combine_body,
        grid=(nb,),
        in_specs=[
            pl.BlockSpec((NC, bs, H), lambda n: (0, n, 0)),
            pl.BlockSpec((3, NC, bs, H), lambda n: (0, 0, n, 0)),
        ],
        out_specs=[
            pl.BlockSpec((bs, H), lambda n: (n, 0)),
            pl.BlockSpec((bs, 3, H), lambda n: (n, 0, 0)),
        ],
        out_shape=[
            jax.ShapeDtypeStruct((N, H), jnp.float32),
            jax.ShapeDtypeStruct((N, 3, H), jnp.float32),
        ],
    )(px, pv)


def kernel(x, vec, edge_index, edge_rbf, edge_vector, W1, b1, W2, b2, We, be):
    f32 = jnp.float32
    j = edge_index[0]
    i = edge_index[1]

    b1r = b1[None, :]
    b2r = b2[None, :]
    ber = be[None, :]

    apk, xh3, v0, v1, v2 = _compute_tables(x, vec, W1, b1r, W2, b2r)
    bpk, rb2, e0, e1, e2 = _compute_edge_arrays(edge_rbf, We, ber,
                                                edge_vector)

    px, pv = _sc_passes(apk, xh3, v0, v1, v2, bpk, rb2, e0, e1, e2, j, i)

    d_x, d_vec = _combine(px, pv)
    return (d_x, d_vec)


# speedup vs baseline: 1.8103x; 1.0026x over previous
import math

import jax
import jax.numpy as jnp
from jax import lax
from jax.experimental import pallas as pl
from jax.experimental.pallas import tpu as pltpu
from jax.experimental.pallas import tpu_sc as plsc

N = 10000
E = 320000
H = 128
INV_SQRT_3 = 1.0 / math.sqrt(3.0)
INV_SQRT_H = 1.0 / math.sqrt(float(H))
SILU_SCALE = 1.0 / 0.6

NC, NS = 2, 16
NW = NC * NS
EPW = E // NW
C = 40
NCH = EPW // C
NPAD = 10240
RPS = NPAD // NS
ZR = C
NZ = RPS // ZR


def _pack_pair(hi, lo):
    hb = lax.bitcast_convert_type(hi, jnp.uint32)
    lb = lax.bitcast_convert_type(lo, jnp.uint32)
    hb = (hb + jnp.uint32(0x8000)) & jnp.uint32(0xFFFF0000)
    lb = (lb + jnp.uint32(0x8000)) >> jnp.uint32(16)
    return lax.bitcast_convert_type(hb | lb, jnp.int32)


_DNUM = (((1,), (1,)), ((), ()))


def _xh_body(x_ref, vec_ref, w1_ref, b1_ref, w2_ref, b2_ref,
             a_ref, o3_ref, v0_ref, v1_ref, v2_ref):
    h = lax.dot_general(x_ref[...], w1_ref[...], _DNUM,
                        preferred_element_type=jnp.float32)
    h = h + b1_ref[...]
    h = h * jax.nn.sigmoid(h) * SILU_SCALE
    o = lax.dot_general(h, w2_ref[...], _DNUM,
                        preferred_element_type=jnp.float32) + b2_ref[...]
    a_ref[...] = _pack_pair(o[:, :H], o[:, H:2 * H])
    o3_ref[...] = lax.bitcast_convert_type(o[:, 2 * H:], jnp.int32)
    v0_ref[...] = vec_ref[:, 0, :]
    v1_ref[...] = vec_ref[:, 1, :]
    v2_ref[...] = vec_ref[:, 2, :]


def _compute_tables(x, vec, w1, b1r, w2, b2r):
    nb = 10
    bs = N // nb
    return pl.pallas_call(
        _xh_body,
        grid=(nb,),
        in_specs=[
            pl.BlockSpec((bs, H), lambda n: (n, 0)),
            pl.BlockSpec((bs, 3, H), lambda n: (n, 0, 0)),
            pl.BlockSpec((H // 2, H), lambda n: (0, 0)),
            pl.BlockSpec((1, H // 2), lambda n: (0, 0)),
            pl.BlockSpec((3 * H, H // 2), lambda n: (0, 0)),
            pl.BlockSpec((1, 3 * H), lambda n: (0, 0)),
        ],
        out_specs=[pl.BlockSpec((bs, H), lambda n: (n, 0))] * 5,
        out_shape=[jax.ShapeDtypeStruct((N, H), jnp.int32)] * 2
        + [jax.ShapeDtypeStruct((N, H), jnp.float32)] * 3,
    )(x, vec, w1, b1r, w2, b2r)


def _rbf_body(rbf_ref, we_ref, be_ref, ev_ref,
              b_ref, r2_ref, e0_ref, e1_ref, e2_ref):
    o = lax.dot_general(rbf_ref[...], we_ref[...], _DNUM,
                        preferred_element_type=jnp.float32)
    o = o + be_ref[...]
    s01 = INV_SQRT_3 * INV_SQRT_H
    o = o * jnp.concatenate([
        jnp.full((1, 2 * H), s01, jnp.float32),
        jnp.full((1, H), INV_SQRT_3, jnp.float32)], axis=1)
    b_ref[...] = _pack_pair(o[:, :H], o[:, H:2 * H])
    r2_ref[...] = lax.bitcast_convert_type(o[:, 2 * H:], jnp.int32)
    e0_ref[...] = jnp.broadcast_to(ev_ref[:, 0][:, None], e0_ref.shape)
    e1_ref[...] = jnp.broadcast_to(ev_ref[:, 1][:, None], e1_ref.shape)
    e2_ref[...] = jnp.broadcast_to(ev_ref[:, 2][:, None], e2_ref.shape)


def _compute_edge_arrays(edge_rbf, we, ber, edge_vector):
    nb = 40
    bs = E // nb
    f = edge_rbf.shape[1]
    return pl.pallas_call(
        _rbf_body,
        grid=(nb,),
        in_specs=[
            pl.BlockSpec((bs, f), lambda n: (n, 0)),
            pl.BlockSpec((3 * H, f), lambda n: (0, 0)),
            pl.BlockSpec((1, 3 * H), lambda n: (0, 0)),
            pl.BlockSpec((bs, 3), lambda n: (n, 0)),
        ],
        out_specs=[pl.BlockSpec((bs, H), lambda n: (n, 0))] * 2
        + [pl.BlockSpec((bs, 16), lambda n: (n, 0))] * 3,
        out_shape=[jax.ShapeDtypeStruct((E, H), jnp.int32)] * 2
        + [jax.ShapeDtypeStruct((E, 16), jnp.float32)] * 3,
    )(edge_rbf, we, ber, edge_vector)


def _zero_acc(zv, acc_sh, sid):
    def zrow(e, _):
        for f in range(H // 16):
            zv[e, pl.ds(f * 16, 16)] = jnp.zeros((16,), jnp.float32)
        return 0
    lax.fori_loop(0, ZR, zrow, 0)

    def zcopy(t, _):
        pltpu.sync_copy(zv, acc_sh.at[pl.ds(sid * RPS + t * ZR, ZR)])
        return 0
    lax.fori_loop(0, NZ, zcopy, 0)


def _writeout(zv, acc_sh, out_hbm, cid, sid):
    def wcopy(t, _):
        r0 = sid * RPS + t * ZR
        pltpu.sync_copy(acc_sh.at[pl.ds(r0, ZR)], zv)
        pltpu.sync_copy(zv, out_hbm.at[cid, pl.ds(r0, ZR)])
        return 0
    lax.fori_loop(0, NZ, wcopy, 0)


MASK_HI = -65536


def _unpack_hi(v):
    return lax.bitcast_convert_type(v & MASK_HI, jnp.float32)


def _unpack_lo(v):
    return lax.bitcast_convert_type(v << 16, jnp.float32)


def _f32(v):
    return lax.bitcast_convert_type(v, jnp.float32)


def _sc_body(apk_h, xh3_h, v0_h, v1_h, v2_h, bpk_h, rb2_h, e0_h, e1_h, e2_h,
             j_h, i_h, px_h, pv_h,
             acc_sh, jv4, iv4, avs, vvs, bvs, evs, sem0, sem1, semi0, semi1):
    cid = lax.axis_index("c")
    sid = lax.axis_index("s")
    wid = sid * NC + cid
    base = wid * EPW
    sems = (sem0, sem1)
    semi = (semi0, semi1)
    zv = vvs.at[0]

    def run_phase(a_h, v_h, b_h, e_h, is_dx, wout):
        _zero_acc(zv, acc_sh, sid)
        plsc.subcore_barrier()

        def fire_idx(t, par):
            d4 = lax.rem(t, 4)
            pltpu.async_copy(j_h.at[pl.ds(base + t * C, C)], jv4.at[d4],
                             semi[par])
            pltpu.async_copy(i_h.at[pl.ds(base + t * C, C)], iv4.at[d4],
                             semi[par])

        def drain_idx(par):
            pltpu.make_async_copy(j_h.at[pl.ds(0, C)], jv4.at[0],
                                  semi[par]).wait()
            pltpu.make_async_copy(i_h.at[pl.ds(0, C)], iv4.at[0],
                                  semi[par]).wait()

        def fire_data(t, b):
            jref = jv4.at[lax.rem(t, 4)]
            off = base + t * C
            pltpu.async_copy(a_h.at[jref], avs.at[b], sems[b])
            pltpu.async_copy(b_h.at[pl.ds(off, C)], bvs.at[b], sems[b])
            if not is_dx:
                pltpu.async_copy(v_h.at[jref], vvs.at[b], sems[b])
                pltpu.async_copy(e_h.at[pl.ds(off, C)], evs.at[b], sems[b])

        def drain_data(b):
            pltpu.make_async_copy(a_h.at[jv4.at[0]], avs.at[b],
                                  sems[b]).wait()
            pltpu.make_async_copy(b_h.at[pl.ds(0, C)], bvs.at[b],
                                  sems[b]).wait()
            if not is_dx:
                pltpu.make_async_copy(v_h.at[jv4.at[0]], vvs.at[b],
                                      sems[b]).wait()
                pltpu.make_async_copy(e_h.at[pl.ds(0, C)], evs.at[b],
                                      sems[b]).wait()

        fire_idx(0, 0)
        fire_idx(1, 1)
        drain_idx(0)
        fire_data(0, 0)
        fire_idx(2, 0)
        drain_idx(1)
        fire_data(1, 1)
        fire_idx(3, 1)

        def step(t, b):
            drain_data(b)
            av_b = avs.at[b]
            vv_b = vvs.at[b]
            bv_b = bvs.at[b]
            ev_b = evs.at[b]

            if is_dx:
                def edge(e, _):
                    for f in range(H // 16):
                        s = pl.ds(f * 16, 16)
                        vv_b[e, s] = _f32(av_b[e, s]) * _f32(bv_b[e, s])
                    return 0
            else:
                def edge(e, _):
                    evr = ev_b[e, pl.ds(0, 16)]
                    for f in range(H // 16):
                        s = pl.ds(f * 16, 16)
                        pa = av_b[e, s]
                        pb = bv_b[e, s]
                        vv_b[e, s] = (_unpack_hi(pa) * _unpack_hi(pb)
                                      * vv_b[e, s]
                                      + _unpack_lo(pa) * _unpack_lo(pb) * evr)
                    return 0
            lax.fori_loop(0, C, edge, 0)
            pltpu.sync_copy(vv_b, acc_sh.at[iv4.at[lax.rem(t, 4)]], add=True)

            @pl.when(t + 2 < NCH)
            def _():
                drain_idx(b)
                fire_data(t + 2, b)

            @pl.when(t + 4 < NCH)
            def _():
                fire_idx(t + 4, b)

        def chunk2(t2, _):
            step(t2 * 2, 0)
            step(t2 * 2 + 1, 1)
            return 0

        lax.fori_loop(0, NCH // 2, chunk2, 0)
        plsc.subcore_barrier()

        def wcopy(t, _):
            r0 = sid * RPS + t * ZR
            pltpu.sync_copy(acc_sh.at[pl.ds(r0, ZR)], zv)
            pltpu.sync_copy(zv, wout(r0))
            return 0
        lax.fori_loop(0, NZ, wcopy, 0)
        plsc.subcore_barrier()

    run_phase(xh3_h, None, rb2_h, None, True,
              lambda r0: px_h.at[cid, pl.ds(r0, ZR)])
    for d, (vh, eh) in enumerate([(v0_h, e0_h), (v1_h, e1_h), (v2_h, e2_h)]):
        run_phase(apk_h, vh, bpk_h, eh, False,
                  lambda r0, d=d: pv_h.at[d, cid, pl.ds(r0, ZR)])


def _sc_passes(apk, xh3, v0, v1, v2, bpk, rb2, e0, e1, e2, j, i):
    mesh = plsc.VectorSubcoreMesh(core_axis_name="c", subcore_axis_name="s")
    return pl.kernel(
        _sc_body,
        out_type=[
            jax.ShapeDtypeStruct((NC, NPAD, H), jnp.float32),
            jax.ShapeDtypeStruct((3, NC, NPAD, H), jnp.float32),
        ],
        mesh=mesh,
        scratch_types=[
            pltpu.VMEM_SHARED((NPAD, H), jnp.float32),
            pltpu.VMEM((4, C), jnp.int32),
            pltpu.VMEM((4, C), jnp.int32),
            pltpu.VMEM((2, C, H), jnp.int32),
            pltpu.VMEM((2, C, H), jnp.float32),
            pltpu.VMEM((2, C, H), jnp.int32),
            pltpu.VMEM((2, C, 16), jnp.float32),
            pltpu.SemaphoreType.DMA,
            pltpu.SemaphoreType.DMA,
            pltpu.SemaphoreType.DMA,
            pltpu.SemaphoreType.DMA,
        ],
    )(apk, xh3, v0, v1, v2, bpk, rb2, e0, e1, e2, j, i)


def _combine_body(px_ref, pv_ref, dx_ref, dv_ref):
    dx_ref[...] = px_ref[0] + px_ref[1]
    dv_ref[...] = jnp.concatenate(
        [(pv_ref[0, 0] + pv_ref[0, 1])[:, None, :],
         (pv_ref[1, 0] + pv_ref[1, 1])[:, None, :],
         (pv_ref[2, 0] + pv_ref[2, 1])[:, None, :]], axis=1)


def _combine(px, pv):
    nb = 25
    bs = N // nb
    return pl.pallas_call(
        _combine_body,
        grid=(nb,),
        in_specs=[
            pl.BlockSpec((NC, bs, H), lambda n: (0, n, 0)),
            pl.BlockSpec((3, NC, bs, H), lambda n: (0, 0, n, 0)),
        ],
        out_specs=[
            pl.BlockSpec((bs, H), lambda n: (n, 0)),
            pl.BlockSpec((bs, 3, H), lambda n: (n, 0, 0)),
        ],
        out_shape=[
            jax.ShapeDtypeStruct((N, H), jnp.float32),
            jax.ShapeDtypeStruct((N, 3, H), jnp.float32),
        ],
    )(px, pv)


def kernel(x, vec, edge_index, edge_rbf, edge_vector, W1, b1, W2, b2, We, be):
    f32 = jnp.float32
    j = edge_index[0]
    i = edge_index[1]

    b1r = b1[None, :]
    b2r = b2[None, :]
    ber = be[None, :]

    apk, xh3, v0, v1, v2 = _compute_tables(x, vec, W1, b1r, W2, b2r)
    bpk, rb2, e0, e1, e2 = _compute_edge_arrays(edge_rbf, We, ber,
                                                edge_vector)

    px, pv = _sc_passes(apk, xh3, v0, v1, v2, bpk, rb2, e0, e1, e2, j, i)

    d_x, d_vec = _combine(px, pv)
    return (d_x, d_vec)
